# baseline (speedup 1.0000x reference)
import functools

import jax
import jax.numpy as jnp
from jax import lax
from jax.experimental import pallas as pl
from jax.experimental.pallas import tpu as pltpu
from jax.experimental.pallas import tpu_sc as plsc

T = 2048
D = 1024
F = 768
E = 8
TOP_K = 2
N = T * TOP_K

NC, NS = 2, 16
NW = NC * NS
L = 16

TMR = 256
TMR_LOG = 8
P = N + E * TMR
NT = P // TMR
NT_PAD = 32

CHUNK = N // NW


def _router_body(x_ref, gate_ref, eids_ref, tw_ref):
    lt = jax.lax.dot_general(gate_ref[...].astype(jnp.bfloat16),
                             x_ref[...].astype(jnp.bfloat16),
                             (((1,), (1,)), ((), ())),
                             preferred_element_type=jnp.float32)
    iota = jax.lax.broadcasted_iota(jnp.int32, lt.shape, 0)
    v1 = jnp.max(lt, axis=0, keepdims=True)
    i1 = jnp.min(jnp.where(lt == v1, iota, E), axis=0, keepdims=True)
    hot1 = iota == i1
    l2 = jnp.where(hot1, jnp.float32(-1e30), lt)
    v2 = jnp.max(l2, axis=0, keepdims=True)
    i2 = jnp.min(jnp.where(l2 == v2, iota, E), axis=0, keepdims=True)
    w1 = jax.nn.sigmoid(v1 - v2)
    eids_ref[...] = jnp.concatenate([i1, i2], axis=0)
    tw_ref[...] = jnp.concatenate([w1, 1.0 - w1], axis=0)


def _router(x, gate_w):
    tma = 512
    return pl.pallas_call(
        _router_body,
        grid=(T // tma,),
        in_specs=[
            pl.BlockSpec((tma, D), lambda t: (t, 0)),
            pl.BlockSpec((E, D), lambda t: (0, 0)),
        ],
        out_specs=[
            pl.BlockSpec((TOP_K, tma), lambda t: (0, t)),
            pl.BlockSpec((TOP_K, tma), lambda t: (0, t)),
        ],
        out_shape=[
            jax.ShapeDtypeStruct((TOP_K, T), jnp.int32),
            jax.ShapeDtypeStruct((TOP_K, T), jnp.float32),
        ],
    )(x, gate_w)


CHUNK_S = N // NS
SUBCH = CHUNK_S // NC


def _dispatch_body(eids_hbm, x_hbm, dst_hbm, xs_hbm, te_hbm,
                   eids_v, hist_all_v, dst_v, te_v, xbuf_v, sem, sem2):
    cid = lax.axis_index("c")
    sid = lax.axis_index("s")
    lane = jax.lax.broadcasted_iota(jnp.int32, (L,), 0)

    pltpu.sync_copy(eids_hbm, eids_v)

    def chunk_hist(c, carry):
        hist = jnp.zeros((L,), jnp.int32)
        for v in range(CHUNK_S // L):
            ev = eids_v[pl.ds(c * CHUNK_S + v * L, L)]
            for e in range(E):
                pc = jnp.sum(jnp.where(ev == e, 1, 0))
                hist = hist + jnp.where(lane == e, pc, 0)
        hist_all_v[c] = hist
        return carry

    lax.fori_loop(0, NS, chunk_hist, 0)

    tot = jnp.zeros((L,), jnp.int32)
    for s in range(NS):
        tot = tot + hist_all_v[s]
    padded = ((tot + (TMR - 1)) >> TMR_LOG) << TMR_LOG
    offs = plsc.cumsum(padded) - padded
    base = offs
    for s in range(NS):
        base = base + jnp.where(jnp.int32(s) < sid, hist_all_v[s], 0)

    @pl.when((sid == 0) & (cid == 0))
    def _tiles():
        for g in range(NT_PAD // L):
            pos = (lane + g * L) * TMR
            te = jnp.zeros((L,), jnp.int32)
            for e in range(1, E):
                oe = jnp.sum(jnp.where(lane == e, offs, 0))
                te = te + jnp.where(pos >= oe, 1, 0)
            te_v[pl.ds(g * L, L)] = te
        pltpu.sync_copy(te_v, te_hbm)

    for v in range(CHUNK_S // L):
        ev = eids_v[pl.ds(sid * CHUNK_S + v * L, L)]
        dst = jnp.zeros((L,), jnp.int32)
        for e in range(E):
            m = ev == e
            mi = jnp.where(m, 1, 0)
            c = plsc.cumsum(mi)
            b_e = jnp.sum(jnp.where(lane == e, base, 0))
            dst = jnp.where(m, b_e + c - 1, dst)
            base = base + jnp.where(lane == e, jnp.sum(mi), 0)
        dst_v[v // 2, pl.ds((v % 2) * L, L)] = dst

    @pl.when(cid == 0)
    def _dst_out():
        pltpu.sync_copy(dst_v, dst_hbm.at[pl.ds(8 * sid, 8)])

    j0 = sid * CHUNK_S + cid * SUBCH
    t0 = j0 % T
    rows = 32
    nch = SUBCH // rows
    reads = [None] * nch

    def rstart(h):
        reads[h] = pltpu.async_copy(
            x_hbm.at[pl.ds(t0 + h * rows, rows)], xbuf_v.at[h % 2], sem2)

    rstart(0)
    for h in range(nch):
        reads[h].wait()
        if h + 1 < nch:
            rstart(h + 1)
        pltpu.async_copy(xbuf_v.at[h % 2],
                         xs_hbm.at[dst_v.at[nch * cid + h]], sem).wait()


def _dispatch(eids_flat, x):
    mesh = plsc.VectorSubcoreMesh(core_axis_name="c", subcore_axis_name="s",
                                  num_cores=NC, num_subcores=NS)
    f = pl.kernel(
        _dispatch_body,
        out_type=[
            jax.ShapeDtypeStruct((N // 32, 32), jnp.int32),
            jax.ShapeDtypeStruct((P, D), jnp.float32),
            jax.ShapeDtypeStruct((NT_PAD,), jnp.int32),
        ],
        mesh=mesh,
        scratch_types=[
            pltpu.VMEM((N,), jnp.int32),
            pltpu.VMEM((NS, L), jnp.int32),
            pltpu.VMEM((CHUNK_S // 32, 32), jnp.int32),
            pltpu.VMEM((NT_PAD,), jnp.int32),
            pltpu.VMEM((2, 32, D), jnp.float32),
            pltpu.SemaphoreType.DMA,
            pltpu.SemaphoreType.DMA,
        ],
        compiler_params=pltpu.CompilerParams(needs_layout_passes=False),
    )
    return f(eids_flat, x)


def _moe_mm_body(te_ref, x_ref, wg_ref, wu_ref, wd_ref, y_ref):
    x = x_ref[...]
    g = jax.lax.dot_general(x, wg_ref[0], (((1,), (1,)), ((), ())),
                            preferred_element_type=jnp.float32)
    u = jax.lax.dot_general(x, wu_ref[0], (((1,), (1,)), ((), ())),
                            preferred_element_type=jnp.float32)
    h = (g * jax.nn.sigmoid(g)) * u
    y_ref[...] = jax.lax.dot_general(h, wd_ref[0], (((1,), (1,)), ((), ())),
                                     preferred_element_type=jnp.float32)


def _grouped_mm(tile_e, x_sorted, w_gate, w_up, w_down):
    grid_spec = pltpu.PrefetchScalarGridSpec(
        num_scalar_prefetch=1,
        grid=(NT,),
        in_specs=[
            pl.BlockSpec((TMR, D), lambda i, te: (i, 0)),
            pl.BlockSpec((1, F, D), lambda i, te: (te[i], 0, 0)),
            pl.BlockSpec((1, F, D), lambda i, te: (te[i], 0, 0)),
            pl.BlockSpec((1, D, F), lambda i, te: (te[i], 0, 0)),
        ],
        out_specs=pl.BlockSpec((TMR, D), lambda i, te: (i, 0)),
    )
    return pl.pallas_call(
        _moe_mm_body,
        grid_spec=grid_spec,
        out_shape=jax.ShapeDtypeStruct((P, D), jnp.float32),
    )(tile_e, x_sorted, w_gate, w_up, w_down)


TOK_PER_W = T // NW
TOK_CHUNK = 16


def _combine_body(y_hbm, dst_hbm, tw_hbm, out_hbm,
                  idx0_v, idx1_v, w0_v, w1_v, rows0_v, rows1_v, obuf_v, sem):
    wid = lax.axis_index("s") * NC + lax.axis_index("c")
    t0 = wid * TOK_PER_W
    pltpu.sync_copy(dst_hbm.at[pl.ds(t0, TOK_PER_W)], idx0_v)
    pltpu.sync_copy(dst_hbm.at[pl.ds(T + t0, TOK_PER_W)], idx1_v)
    pltpu.sync_copy(tw_hbm.at[pl.ds(t0, TOK_PER_W)], w0_v)
    pltpu.sync_copy(tw_hbm.at[pl.ds(T + t0, TOK_PER_W)], w1_v)
    lane = jax.lax.broadcasted_iota(jnp.int32, (L,), 0)
    nch = TOK_PER_W // TOK_CHUNK
    cps = [None] * nch

    def start(c):
        b = c % 2
        cp0 = pltpu.async_copy(
            y_hbm.at[idx0_v.at[pl.ds(c * TOK_CHUNK, TOK_CHUNK)]],
            rows0_v.at[b], sem)
        cp1 = pltpu.async_copy(
            y_hbm.at[idx1_v.at[pl.ds(c * TOK_CHUNK, TOK_CHUNK)]],
            rows1_v.at[b], sem)
        cps[c] = (cp0, cp1)

    start(0)
    for c in range(nch):
        cps[c][0].wait()
        cps[c][1].wait()
        if c + 1 < nch:
            start(c + 1)
        b = c % 2
        w0c = w0_v[pl.ds(c * TOK_CHUNK, TOK_CHUNK)]
        w1c = w1_v[pl.ds(c * TOK_CHUNK, TOK_CHUNK)]

        def tok_body(i, carry):
            w0s = jnp.sum(jnp.where(lane == i, w0c, 0.0))
            w1s = jnp.sum(jnp.where(lane == i, w1c, 0.0))
            for dchunk in range(D // L):
                sl = pl.ds(dchunk * L, L)
                obuf_v[i, sl] = (w0s * rows0_v[b, i, sl]
                                 + w1s * rows1_v[b, i, sl])
            return carry

        lax.fori_loop(0, TOK_CHUNK, tok_body, 0)
        pltpu.sync_copy(obuf_v, out_hbm.at[pl.ds(t0 + c * TOK_CHUNK,
                                                 TOK_CHUNK)])


def _combine(y_sorted, dst_flat, tw_flat):
    mesh = plsc.VectorSubcoreMesh(core_axis_name="c", subcore_axis_name="s",
                                  num_cores=NC, num_subcores=NS)
    f = pl.kernel(
        _combine_body,
        out_type=jax.ShapeDtypeStruct((T, D), jnp.float32),
        mesh=mesh,
        scratch_types=[
            pltpu.VMEM((TOK_PER_W,), jnp.int32),
            pltpu.VMEM((TOK_PER_W,), jnp.int32),
            pltpu.VMEM((TOK_PER_W,), jnp.float32),
            pltpu.VMEM((TOK_PER_W,), jnp.float32),
            pltpu.VMEM((2, TOK_CHUNK, D), jnp.float32),
            pltpu.VMEM((2, TOK_CHUNK, D), jnp.float32),
            pltpu.VMEM((TOK_CHUNK, D), jnp.float32),
            pltpu.SemaphoreType.DMA,
        ],
        compiler_params=pltpu.CompilerParams(needs_layout_passes=False),
    )
    return f(y_sorted, dst_flat, tw_flat)


def kernel(hidden_states, gate_w, w_gate, w_up, w_down):
    orig_shape = hidden_states.shape
    x = hidden_states.reshape(-1, orig_shape[-1])
    eids, tw = _router(x, gate_w)
    dst2d, x_sorted, tile_e = _dispatch(eids.reshape(N), x)
    y_sorted = _grouped_mm(tile_e, x_sorted, w_gate, w_up, w_down)
    y_sorted = _grouped_mm(tile_e, x_sorted,
                           w_gate.astype(jnp.bfloat16),
                           w_up.astype(jnp.bfloat16),
                           w_down.astype(jnp.bfloat16))
    out = _combine(y_sorted, dst2d.reshape(N), tw.reshape(N))
    return out.reshape(orig_shape)


# speedup vs baseline: 1.0224x; 1.0224x over previous
import functools

import jax
import jax.numpy as jnp
from jax import lax
from jax.experimental import pallas as pl
from jax.experimental.pallas import tpu as pltpu
from jax.experimental.pallas import tpu_sc as plsc

T = 2048
D = 1024
F = 768
E = 8
TOP_K = 2
N = T * TOP_K

NC, NS = 2, 16
NW = NC * NS
L = 16

TMR = 256
TMR_LOG = 8
P = N + E * TMR
NT = P // TMR
NT_PAD = 32

CHUNK = N // NW


def _router_body(x_ref, gate_ref, eids_ref, tw_ref):
    lt = jax.lax.dot_general(gate_ref[...].astype(jnp.bfloat16),
                             x_ref[...].astype(jnp.bfloat16),
                             (((1,), (1,)), ((), ())),
                             preferred_element_type=jnp.float32)
    iota = jax.lax.broadcasted_iota(jnp.int32, lt.shape, 0)
    v1 = jnp.max(lt, axis=0, keepdims=True)
    i1 = jnp.min(jnp.where(lt == v1, iota, E), axis=0, keepdims=True)
    hot1 = iota == i1
    l2 = jnp.where(hot1, jnp.float32(-1e30), lt)
    v2 = jnp.max(l2, axis=0, keepdims=True)
    i2 = jnp.min(jnp.where(l2 == v2, iota, E), axis=0, keepdims=True)
    w1 = jax.nn.sigmoid(v1 - v2)
    eids_ref[...] = jnp.concatenate([i1, i2], axis=0)
    tw_ref[...] = jnp.concatenate([w1, 1.0 - w1], axis=0)


def _router(x, gate_w):
    tma = 512
    return pl.pallas_call(
        _router_body,
        grid=(T // tma,),
        in_specs=[
            pl.BlockSpec((tma, D), lambda t: (t, 0)),
            pl.BlockSpec((E, D), lambda t: (0, 0)),
        ],
        out_specs=[
            pl.BlockSpec((TOP_K, tma), lambda t: (0, t)),
            pl.BlockSpec((TOP_K, tma), lambda t: (0, t)),
        ],
        out_shape=[
            jax.ShapeDtypeStruct((TOP_K, T), jnp.int32),
            jax.ShapeDtypeStruct((TOP_K, T), jnp.float32),
        ],
    )(x, gate_w)


CHUNK_S = N // NS
SUBCH = CHUNK_S // NC


def _dispatch_body(eids_hbm, x_hbm, dst_hbm, xs_hbm, te_hbm,
                   eids_v, hist_all_v, dst_v, te_v, xbuf_v, sem, sem2):
    cid = lax.axis_index("c")
    sid = lax.axis_index("s")
    lane = jax.lax.broadcasted_iota(jnp.int32, (L,), 0)

    pltpu.sync_copy(eids_hbm, eids_v)

    def chunk_hist(c, carry):
        hist = jnp.zeros((L,), jnp.int32)
        for v in range(CHUNK_S // L):
            ev = eids_v[pl.ds(c * CHUNK_S + v * L, L)]
            for e in range(E):
                pc = jnp.sum(jnp.where(ev == e, 1, 0))
                hist = hist + jnp.where(lane == e, pc, 0)
        hist_all_v[c] = hist
        return carry

    lax.fori_loop(0, NS, chunk_hist, 0)

    tot = jnp.zeros((L,), jnp.int32)
    for s in range(NS):
        tot = tot + hist_all_v[s]
    padded = ((tot + (TMR - 1)) >> TMR_LOG) << TMR_LOG
    offs = plsc.cumsum(padded) - padded
    base = offs
    for s in range(NS):
        base = base + jnp.where(jnp.int32(s) < sid, hist_all_v[s], 0)

    @pl.when((sid == 0) & (cid == 0))
    def _tiles():
        tp = jnp.sum(padded)
        for g in range(NT_PAD // L):
            pos = (lane + g * L) * TMR
            te = jnp.zeros((L,), jnp.int32)
            for e in range(1, E):
                oe = jnp.sum(jnp.where(lane == e, offs, 0))
                te = te + jnp.where(pos >= oe, 1, 0)
            te_v[0, pl.ds(g * L, L)] = te
            te_v[1, pl.ds(g * L, L)] = jnp.where(pos < tp, 1, 0)
        pltpu.sync_copy(te_v, te_hbm)

    for v in range(CHUNK_S // L):
        ev = eids_v[pl.ds(sid * CHUNK_S + v * L, L)]
        dst = jnp.zeros((L,), jnp.int32)
        for e in range(E):
            m = ev == e
            mi = jnp.where(m, 1, 0)
            c = plsc.cumsum(mi)
            b_e = jnp.sum(jnp.where(lane == e, base, 0))
            dst = jnp.where(m, b_e + c - 1, dst)
            base = base + jnp.where(lane == e, jnp.sum(mi), 0)
        dst_v[v // 2, pl.ds((v % 2) * L, L)] = dst

    @pl.when(cid == 0)
    def _dst_out():
        pltpu.sync_copy(dst_v, dst_hbm.at[pl.ds(8 * sid, 8)])

    j0 = sid * CHUNK_S + cid * SUBCH
    t0 = j0 % T
    rows = 32
    nch = SUBCH // rows
    reads = [None] * nch

    def rstart(h):
        reads[h] = pltpu.async_copy(
            x_hbm.at[pl.ds(t0 + h * rows, rows)], xbuf_v.at[h % 2], sem2)

    rstart(0)
    for h in range(nch):
        reads[h].wait()
        if h + 1 < nch:
            rstart(h + 1)
        pltpu.async_copy(xbuf_v.at[h % 2],
                         xs_hbm.at[dst_v.at[nch * cid + h]], sem).wait()


def _dispatch(eids_flat, x):
    mesh = plsc.VectorSubcoreMesh(core_axis_name="c", subcore_axis_name="s",
                                  num_cores=NC, num_subcores=NS)
    f = pl.kernel(
        _dispatch_body,
        out_type=[
            jax.ShapeDtypeStruct((N // 32, 32), jnp.int32),
            jax.ShapeDtypeStruct((P, D), jnp.float32),
            jax.ShapeDtypeStruct((2, NT_PAD), jnp.int32),
        ],
        mesh=mesh,
        scratch_types=[
            pltpu.VMEM((N,), jnp.int32),
            pltpu.VMEM((NS, L), jnp.int32),
            pltpu.VMEM((CHUNK_S // 32, 32), jnp.int32),
            pltpu.VMEM((2, NT_PAD), jnp.int32),
            pltpu.VMEM((2, 32, D), jnp.float32),
            pltpu.SemaphoreType.DMA,
            pltpu.SemaphoreType.DMA,
        ],
        compiler_params=pltpu.CompilerParams(needs_layout_passes=False),
    )
    return f(eids_flat, x)


def _moe_mm_body(te_ref, x_ref, wg_ref, wu_ref, wd_ref, y_ref):
    i = pl.program_id(0)

    @pl.when(te_ref[1, i] != 0)
    def _():
        x = x_ref[...]
        g = jax.lax.dot_general(x, wg_ref[0], (((1,), (1,)), ((), ())),
                                preferred_element_type=jnp.float32)
        u = jax.lax.dot_general(x, wu_ref[0], (((1,), (1,)), ((), ())),
                                preferred_element_type=jnp.float32)
        h = (g * jax.nn.sigmoid(g)) * u
        y_ref[...] = jax.lax.dot_general(h, wd_ref[0],
                                         (((1,), (1,)), ((), ())),
                                         preferred_element_type=jnp.float32)


def _grouped_mm(tile_e, x_sorted, w_gate, w_up, w_down):
    grid_spec = pltpu.PrefetchScalarGridSpec(
        num_scalar_prefetch=1,
        grid=(NT,),
        in_specs=[
            pl.BlockSpec((TMR, D), lambda i, te: (i, 0)),
            pl.BlockSpec((1, F, D), lambda i, te: (te[0, i], 0, 0)),
            pl.BlockSpec((1, F, D), lambda i, te: (te[0, i], 0, 0)),
            pl.BlockSpec((1, D, F), lambda i, te: (te[0, i], 0, 0)),
        ],
        out_specs=pl.BlockSpec((TMR, D), lambda i, te: (i, 0)),
    )
    return pl.pallas_call(
        _moe_mm_body,
        grid_spec=grid_spec,
        out_shape=jax.ShapeDtypeStruct((P, D), jnp.float32),
    )(tile_e, x_sorted, w_gate, w_up, w_down)


TOK_PER_W = T // NW
TOK_CHUNK = 16


def _combine_body(y_hbm, dst_hbm, tw_hbm, out_hbm,
                  idx0_v, idx1_v, w0_v, w1_v, rows0_v, rows1_v, obuf_v, sem):
    wid = lax.axis_index("s") * NC + lax.axis_index("c")
    t0 = wid * TOK_PER_W
    pltpu.sync_copy(dst_hbm.at[pl.ds(t0, TOK_PER_W)], idx0_v)
    pltpu.sync_copy(dst_hbm.at[pl.ds(T + t0, TOK_PER_W)], idx1_v)
    pltpu.sync_copy(tw_hbm.at[pl.ds(t0, TOK_PER_W)], w0_v)
    pltpu.sync_copy(tw_hbm.at[pl.ds(T + t0, TOK_PER_W)], w1_v)
    lane = jax.lax.broadcasted_iota(jnp.int32, (L,), 0)
    nch = TOK_PER_W // TOK_CHUNK
    cps = [None] * nch

    def start(c):
        b = c % 2
        cp0 = pltpu.async_copy(
            y_hbm.at[idx0_v.at[pl.ds(c * TOK_CHUNK, TOK_CHUNK)]],
            rows0_v.at[b], sem)
        cp1 = pltpu.async_copy(
            y_hbm.at[idx1_v.at[pl.ds(c * TOK_CHUNK, TOK_CHUNK)]],
            rows1_v.at[b], sem)
        cps[c] = (cp0, cp1)

    start(0)
    for c in range(nch):
        cps[c][0].wait()
        cps[c][1].wait()
        if c + 1 < nch:
            start(c + 1)
        b = c % 2
        w0c = w0_v[pl.ds(c * TOK_CHUNK, TOK_CHUNK)]
        w1c = w1_v[pl.ds(c * TOK_CHUNK, TOK_CHUNK)]

        def tok_body(i, carry):
            w0s = jnp.sum(jnp.where(lane == i, w0c, 0.0))
            w1s = jnp.sum(jnp.where(lane == i, w1c, 0.0))
            for dchunk in range(D // L):
                sl = pl.ds(dchunk * L, L)
                obuf_v[i, sl] = (w0s * rows0_v[b, i, sl]
                                 + w1s * rows1_v[b, i, sl])
            return carry

        lax.fori_loop(0, TOK_CHUNK, tok_body, 0)
        pltpu.sync_copy(obuf_v, out_hbm.at[pl.ds(t0 + c * TOK_CHUNK,
                                                 TOK_CHUNK)])


def _combine(y_sorted, dst_flat, tw_flat):
    mesh = plsc.VectorSubcoreMesh(core_axis_name="c", subcore_axis_name="s",
                                  num_cores=NC, num_subcores=NS)
    f = pl.kernel(
        _combine_body,
        out_type=jax.ShapeDtypeStruct((T, D), jnp.float32),
        mesh=mesh,
        scratch_types=[
            pltpu.VMEM((TOK_PER_W,), jnp.int32),
            pltpu.VMEM((TOK_PER_W,), jnp.int32),
            pltpu.VMEM((TOK_PER_W,), jnp.float32),
            pltpu.VMEM((TOK_PER_W,), jnp.float32),
            pltpu.VMEM((2, TOK_CHUNK, D), jnp.float32),
            pltpu.VMEM((2, TOK_CHUNK, D), jnp.float32),
            pltpu.VMEM((TOK_CHUNK, D), jnp.float32),
            pltpu.SemaphoreType.DMA,
        ],
        compiler_params=pltpu.CompilerParams(needs_layout_passes=False),
    )
    return f(y_sorted, dst_flat, tw_flat)


def kernel(hidden_states, gate_w, w_gate, w_up, w_down):
    orig_shape = hidden_states.shape
    x = hidden_states.reshape(-1, orig_shape[-1])
    eids, tw = _router(x, gate_w)
    dst2d, x_sorted, tile_e = _dispatch(eids.reshape(N), x)
    y_sorted = _grouped_mm(tile_e, x_sorted, w_gate, w_up, w_down)
    y_sorted = _grouped_mm(tile_e, x_sorted,
                           w_gate.astype(jnp.bfloat16),
                           w_up.astype(jnp.bfloat16),
                           w_down.astype(jnp.bfloat16))
    out = _combine(y_sorted, dst2d.reshape(N), tw.reshape(N))
    return out.reshape(orig_shape)
